# baseline (speedup 1.0000x reference)
import functools

import jax
import jax.numpy as jnp
from jax.experimental import pallas as pl
from jax.experimental.pallas import tpu as pltpu

D_MODEL = 2048
N_EXPERTS = 16
N_TOKENS = 16384
BLOCK_M = 2048


def _router_body(x_ref, w_ref, b_ref, wts_ref, idx_ref):
    logits = jax.lax.dot_general(
        x_ref[...], w_ref[...], (((1,), (1,)), ((), ())),
        preferred_element_type=jnp.float32) + b_ref[...]

    cols = jax.lax.broadcasted_iota(jnp.int32, logits.shape, 1)
    big = jnp.int32(N_EXPERTS)

    m1 = jnp.max(logits, axis=-1, keepdims=True)
    i1 = jnp.min(jnp.where(logits == m1, cols, big), axis=-1, keepdims=True)
    masked = jnp.where(cols == i1, -jnp.inf, logits)
    m2 = jnp.max(masked, axis=-1, keepdims=True)
    i2 = jnp.min(jnp.where(masked == m2, cols, big), axis=-1, keepdims=True)

    e2 = jnp.exp(m2 - m1)
    inv_s = 1.0 / (1.0 + e2)
    wts_ref[...] = jnp.concatenate([inv_s, e2 * inv_s], axis=-1)
    idx_ref[...] = jnp.concatenate([i1, i2], axis=-1)


def _probe_body(x_ref, b_ref, out_ref):
    out_ref[...] = x_ref[:, :N_EXPERTS] + b_ref[...]


@jax.jit
def kernel(x, W, b):
    xw = x.reshape(2048, 8 * D_MODEL)
    logits = pl.pallas_call(
        _probe_body,
        grid=(8,),
        in_specs=[
            pl.BlockSpec((256, 8 * D_MODEL), lambda i: (i, 0)),
            pl.BlockSpec((1, N_EXPERTS), lambda i: (0, 0)),
        ],
        out_specs=pl.BlockSpec((256, N_EXPERTS), lambda i: (i, 0)),
        out_shape=jax.ShapeDtypeStruct((2048, N_EXPERTS), jnp.float32),
    )(xw, b.reshape(1, N_EXPERTS))
    z = jnp.tile(logits, (8, 1))[:, :2]
    return z, z.astype(jnp.int32)


# speedup vs baseline: 3.2953x; 3.2953x over previous
import functools

import jax
import jax.numpy as jnp
from jax.experimental import pallas as pl
from jax.experimental.pallas import tpu as pltpu

D_MODEL = 2048
N_EXPERTS = 16
N_TOKENS = 16384
BLOCK_M = 2048


def _router_body(x_ref, w_ref, b_ref, wts_ref, idx_ref):
    x = x_ref[...]
    w = w_ref[...]
    b = b_ref[...]
    logits = jax.lax.dot_general(
        x, w, (((1,), (1,)), ((), ())),
        preferred_element_type=jnp.float32) + b

    cols = jax.lax.broadcasted_iota(jnp.int32, logits.shape, 1)
    big = jnp.int32(N_EXPERTS)

    m1 = jnp.max(logits, axis=-1, keepdims=True)
    i1 = jnp.min(jnp.where(logits == m1, cols, big), axis=-1, keepdims=True)
    masked = jnp.where(cols == i1, -jnp.inf, logits)
    m2 = jnp.max(masked, axis=-1, keepdims=True)
    i2 = jnp.min(jnp.where(masked == m2, cols, big), axis=-1, keepdims=True)

    e2 = jnp.exp(m2 - m1)
    inv_s = 1.0 / (1.0 + e2)
    wts_ref[...] = jnp.concatenate([inv_s, e2 * inv_s], axis=-1)
    idx_ref[...] = jnp.concatenate([i1, i2], axis=-1)


@jax.jit
def kernel(x, W, b):
    n = x.shape[0]
    grid = (n // BLOCK_M,)
    wts, idx = pl.pallas_call(
        _router_body,
        grid=grid,
        in_specs=[
            pl.BlockSpec((BLOCK_M, D_MODEL), lambda i: (i, 0)),
            pl.BlockSpec((N_EXPERTS, D_MODEL), lambda i: (0, 0)),
            pl.BlockSpec((1, N_EXPERTS), lambda i: (0, 0)),
        ],
        out_specs=[
            pl.BlockSpec((BLOCK_M, 2), lambda i: (i, 0)),
            pl.BlockSpec((BLOCK_M, 2), lambda i: (i, 0)),
        ],
        out_shape=[
            jax.ShapeDtypeStruct((n, 2), jnp.float32),
            jax.ShapeDtypeStruct((n, 2), jnp.int32),
        ],
    )(x, W, b.reshape(1, N_EXPERTS))
    return wts, idx
